# Initial kernel scaffold; baseline (speedup 1.0000x reference)
import jax
import jax.numpy as jnp
from jax.experimental import pallas as pl


def kernel(idx, table):
    raise NotImplementedError("write your pallas kernel here")


# speedup vs baseline: 1.7008x; 1.7008x over previous
import functools

import jax
import jax.numpy as jnp
from jax import lax
from jax.experimental import pallas as pl
from jax.experimental.pallas import tpu as pltpu
from jax.experimental.pallas import tpu_sc as plsc

_info = plsc.get_sparse_core_info()
_NC, _NS = _info.num_cores, _info.num_subcores
_NW = _NC * _NS


@functools.lru_cache(maxsize=None)
def _make_gather(n, d, dtype_name):
    dtype = jnp.dtype(dtype_name)
    assert n % _NW == 0
    rows_per_w = n // _NW
    chunk = 120
    while rows_per_w % chunk:
        chunk -= 8
    n_chunks = rows_per_w // chunk

    mesh = plsc.VectorSubcoreMesh(core_axis_name="c", subcore_axis_name="s")

    @functools.partial(
        pl.kernel,
        mesh=mesh,
        out_type=jax.ShapeDtypeStruct((n, d), dtype),
        scratch_types=[
            pltpu.VMEM((chunk,), jnp.int32),
            pltpu.VMEM((chunk, d), dtype),
            pltpu.SemaphoreType.DMA,
        ],
    )
    def gather_kernel(idx_hbm, table_hbm, out_hbm, idx_v, rows_v, sem):
        wid = lax.axis_index("s") * _NC + lax.axis_index("c")
        base = wid * rows_per_w

        def body(j, carry):
            off = base + j * chunk
            pltpu.sync_copy(idx_hbm.at[pl.ds(off, chunk)], idx_v)
            pltpu.async_copy(table_hbm.at[idx_v], rows_v, sem).wait()
            pltpu.sync_copy(rows_v, out_hbm.at[pl.ds(off, chunk)])
            return carry

        lax.fori_loop(0, n_chunks, body, 0)

    return gather_kernel


def kernel(idx, table):
    d = table.shape[1]
    idx_flat = idx.reshape(-1)
    out = _make_gather(idx_flat.shape[0], d, table.dtype.name)(idx_flat, table)
    return out.reshape(idx.shape + (d,))


# speedup vs baseline: 1.8909x; 1.1118x over previous
import functools

import jax
import jax.numpy as jnp
from jax import lax
from jax.experimental import pallas as pl
from jax.experimental.pallas import tpu as pltpu
from jax.experimental.pallas import tpu_sc as plsc

_info = plsc.get_sparse_core_info()
_NC, _NS = _info.num_cores, _info.num_subcores
_NW = _NC * _NS

_CHUNK = 112
_NBUF = 3


@functools.lru_cache(maxsize=None)
def _make_gather(n, d, dtype_name):
    dtype = jnp.dtype(dtype_name)
    assert n % (_NW * _CHUNK) == 0
    rows_per_w = n // _NW
    n_chunks = rows_per_w // _CHUNK
    assert n_chunks % _NBUF == 0
    n_grp = n_chunks // _NBUF

    mesh = plsc.VectorSubcoreMesh(core_axis_name="c", subcore_axis_name="s")

    row_buf = pltpu.VMEM((_CHUNK, d), dtype)

    @functools.partial(
        pl.kernel,
        mesh=mesh,
        out_type=jax.ShapeDtypeStruct((n, d), dtype),
        scratch_types=(
            [pltpu.VMEM((n_chunks, _CHUNK), jnp.int32)]
            + [row_buf] * _NBUF
            + [pltpu.SemaphoreType.DMA] * _NBUF
            + [pltpu.SemaphoreType.DMA] * _NBUF
        ),
    )
    def gather_kernel(idx_hbm, table_hbm, out_hbm, idx_v, *bufs_and_sems):
        rows = bufs_and_sems[:_NBUF]
        gsem = bufs_and_sems[_NBUF:2 * _NBUF]
        osem = bufs_and_sems[2 * _NBUF:]
        wid = lax.axis_index("s") * _NC + lax.axis_index("c")
        base = wid * rows_per_w

        pltpu.sync_copy(idx_hbm.at[wid], idx_v)

        def fire_gather(j, b):
            pltpu.async_copy(table_hbm.at[idx_v.at[j]], rows[b], gsem[b])

        def fire_write(j, b):
            pltpu.async_copy(
                rows[b], out_hbm.at[pl.ds(base + j * _CHUNK, _CHUNK)], osem[b])

        def wait_gather(b):
            pltpu.make_async_copy(
                table_hbm.at[pl.ds(0, _CHUNK)], rows[b], gsem[b]).wait()

        def wait_write(b):
            pltpu.make_async_copy(
                rows[b], out_hbm.at[pl.ds(base, _CHUNK)], osem[b]).wait()

        for b in range(_NBUF):
            fire_gather(b, b)

        def body(g, carry):
            j0 = g * _NBUF
            for b in range(_NBUF):
                wait_gather(b)
                fire_write(j0 + b, b)
            for b in range(_NBUF):
                wait_write(b)
                fire_gather(j0 + _NBUF + b, b)
            return carry

        lax.fori_loop(0, n_grp - 1, body, 0)

        j0 = (n_grp - 1) * _NBUF
        for b in range(_NBUF):
            wait_gather(b)
            fire_write(j0 + b, b)
        for b in range(_NBUF):
            wait_write(b)

    return gather_kernel


def kernel(idx, table):
    d = table.shape[1]
    idx_flat = idx.reshape(_NW, -1, _CHUNK)
    out = _make_gather(idx.size, d, table.dtype.name)(idx_flat, table)
    return out.reshape(idx.shape + (d,))
